# baseline (speedup 1.0000x reference)
import functools

import jax
import jax.numpy as jnp
from jax import lax
from jax.experimental import pallas as pl
from jax.experimental.pallas import tpu as pltpu
from jax.experimental.pallas import tpu_sc as plsc

_CH = 384
_B = 32
_MAXF_BITS = 0x7F7FFFFF
_SEL_LANES = 16
_SEL_WORKERS = _CH // _SEL_LANES


def _max_kernel(x_ref, m_ref):
    m_ref[0] = jnp.max(x_ref[...], axis=0, keepdims=True)


def _mid_a_kernel(m_ref, dti_ref, t_ref):
    m = m_ref[...]
    alpha = jax.nn.sigmoid(m)
    beta = 1.0 - alpha

    t = jnp.floor(jnp.sum(jnp.exp(beta - alpha)) / _B).astype(jnp.int32)
    t = jnp.where(t <= 2, 3, t)
    t = jnp.minimum(t, _CH)
    t_ref[...] = jnp.full((1, 128), t, jnp.int32)

    hi = lax.Precision.HIGHEST
    ones_col = jnp.ones((_B, 1), jnp.float32)
    a2_row = jnp.sum(alpha * alpha, axis=0, keepdims=True)
    b2_col = lax.dot_general(beta * beta, ones_col, (((0,), (0,)), ((), ())),
                             precision=hi)
    gt = lax.dot_general(beta, alpha, (((0,), (0,)), ((), ())),
                         precision=hi)
    dt = jnp.sqrt(b2_col + 2.0 * gt + a2_row)
    dti_ref[...] = lax.bitcast_convert_type(dt, jnp.int32)


def _sel_body(dti_hbm, t_hbm, out_hbm, dcols, tv, vtv, stage):
    cc = lax.axis_index("c")
    ss = lax.axis_index("s")
    wid = cc * 16 + ss

    @pl.when(wid < _SEL_WORKERS)
    def _():
        grp = lax.div(wid, 8)
        sub = lax.rem(wid, 8) * _SEL_LANES
        pltpu.sync_copy(dti_hbm.at[:, pl.ds(grp * 128, 128)], dcols)
        pltpu.sync_copy(t_hbm.at[0, pl.ds(0, _SEL_LANES)], tv)
        t_vec = tv[...]

        def body(_, c):
            lo, hi = c
            mid = lo + lax.shift_right_logical(hi - lo, 1)

            def jbody(j, cnt):
                for u in range(8):
                    col = dcols[j * 8 + u, pl.ds(sub, _SEL_LANES)]
                    cnt = cnt + jnp.where(col <= mid, 1, 0)
                return cnt

            cnt = lax.fori_loop(0, _CH // 8, jbody,
                                jnp.zeros((_SEL_LANES,), jnp.int32))
            ok = cnt >= t_vec
            return jnp.where(ok, lo, mid + 1), jnp.where(ok, mid, hi)

        lo0 = jnp.zeros((_SEL_LANES,), jnp.int32)
        hi0 = jnp.full((_SEL_LANES,), _MAXF_BITS, jnp.int32)
        _, res = lax.fori_loop(0, 31, body, (lo0, hi0))
        vtv[...] = lax.bitcast_convert_type(res, jnp.float32)
        pltpu.sync_copy(vtv, stage.at[pl.ds(ss * _SEL_LANES, _SEL_LANES)])

    plsc.subcore_barrier()

    @pl.when(jnp.logical_and(ss == 0, cc == 0))
    def _():
        pltpu.sync_copy(stage.at[pl.ds(0, 256)], out_hbm.at[pl.ds(0, 256)])

    @pl.when(jnp.logical_and(ss == 0, cc == 1))
    def _():
        pltpu.sync_copy(stage.at[pl.ds(0, 128)], out_hbm.at[pl.ds(256, 128)])


def _select_vt(dti, tvec):
    mesh = plsc.VectorSubcoreMesh(core_axis_name="c", subcore_axis_name="s")
    sel = functools.partial(
        pl.kernel,
        mesh=mesh,
        out_type=jax.ShapeDtypeStruct((_CH,), jnp.float32),
        scratch_types=[
            pltpu.VMEM((_CH, 128), jnp.int32),
            pltpu.VMEM((_SEL_LANES,), jnp.int32),
            pltpu.VMEM((_SEL_LANES,), jnp.float32),
            pltpu.VMEM_SHARED((256,), jnp.float32),
        ],
    )(_sel_body)
    return sel(dti, tvec)


def _mid_b_kernel(m_ref, dti_ref, vt_ref, s_ref):
    m = m_ref[...]
    alpha = jax.nn.sigmoid(m)

    hi = lax.Precision.HIGHEST
    ones_row = jnp.ones((1, _CH), jnp.float32)
    dt = lax.bitcast_convert_type(dti_ref[...], jnp.float32)
    vt = vt_ref[...]

    m1 = jnp.min(dt, axis=0, keepdims=True)
    nmin = jnp.sum((dt == m1).astype(jnp.int32), axis=0, keepdims=True)
    m2 = jnp.min(jnp.where(dt > m1, dt, jnp.inf), axis=0, keepdims=True)
    sigma = jnp.where(nmin > 1, m1, m2)

    rr = dt / sigma
    wt = jnp.where(dt < vt, jnp.exp(-(rr * rr)), 0.0)
    wt2 = wt * wt
    rowsq = jnp.sum(wt2, axis=0, keepdims=True)
    colsq = lax.dot_general(ones_row, wt2, (((1,), (1,)), ((), ())),
                            precision=hi)
    ri = lax.broadcasted_iota(jnp.int32, (_CH, _CH), 0)
    ci = lax.broadcasted_iota(jnp.int32, (_CH, _CH), 1)
    wdiag = jnp.sum(jnp.where(ri == ci, wt, 0.0), axis=0, keepdims=True)
    f2 = colsq + rowsq
    eps = jnp.where(f2 > 0.0, wdiag / jnp.sqrt(jnp.maximum(f2, 1e-38)), 0.0)

    s_ref[...] = alpha * (1.0 + eps)


def _scale_kernel(s_ref, x_ref, o_ref):
    o_ref[...] = x_ref[...] * s_ref[...][0]


def kernel(x):
    b, c, h, w = x.shape
    hw = h * w
    xt = jnp.transpose(x, (0, 2, 3, 1)).reshape(b * hw, c)

    m = pl.pallas_call(
        _max_kernel,
        grid=(b,),
        in_specs=[pl.BlockSpec((hw, c), lambda i: (i, 0))],
        out_specs=pl.BlockSpec((1, 1, c), lambda i: (i, 0, 0)),
        out_shape=jax.ShapeDtypeStruct((b, 1, c), jnp.float32),
    )(xt)
    m = m.reshape(b, c)

    dti, tvec = pl.pallas_call(
        _mid_a_kernel,
        out_shape=(
            jax.ShapeDtypeStruct((c, c), jnp.int32),
            jax.ShapeDtypeStruct((1, 128), jnp.int32),
        ),
    )(m)

    vt = _select_vt(dti, tvec)

    scale = pl.pallas_call(
        _mid_b_kernel,
        out_shape=jax.ShapeDtypeStruct((b, c), jnp.float32),
    )(m, dti, vt.reshape(1, c))

    out = pl.pallas_call(
        _scale_kernel,
        grid=(b,),
        in_specs=[
            pl.BlockSpec((1, 1, c), lambda i: (i, 0, 0)),
            pl.BlockSpec((hw, c), lambda i: (i, 0)),
        ],
        out_specs=pl.BlockSpec((hw, c), lambda i: (i, 0)),
        out_shape=jax.ShapeDtypeStruct((b * hw, c), jnp.float32),
    )(scale.reshape(b, 1, c), xt)

    return jnp.transpose(out.reshape(b, h, w, c), (0, 3, 1, 2))


# speedup vs baseline: 1.0266x; 1.0266x over previous
import jax
import jax.numpy as jnp
from jax import lax
from jax.experimental import pallas as pl

_CH = 384
_B = 32
_MAXF_BITS = 0x7F7FFFFF


def _max_kernel(x_ref, m_ref):
    part = jnp.max(x_ref[...], axis=0, keepdims=True)

    @pl.when(pl.program_id(1) == 0)
    def _():
        m_ref[0] = part

    @pl.when(pl.program_id(1) != 0)
    def _():
        m_ref[0] = jnp.maximum(m_ref[0], part)


def _mid_kernel(m_ref, s_ref):
    m = m_ref[...]
    alpha = jax.nn.sigmoid(m)
    beta = 1.0 - alpha

    t = jnp.floor(jnp.sum(jnp.exp(beta - alpha)) / _B).astype(jnp.int32)
    t = jnp.where(t <= 2, 3, t)
    t = jnp.minimum(t, _CH)

    hi = lax.Precision.HIGHEST
    ones_col = jnp.ones((_B, 1), jnp.float32)
    ones_row = jnp.ones((1, _CH), jnp.float32)
    a2_row = jnp.sum(alpha * alpha, axis=0, keepdims=True)
    b2_col = lax.dot_general(beta * beta, ones_col, (((0,), (0,)), ((), ())),
                             precision=hi)
    gt = lax.dot_general(beta, alpha, (((0,), (0,)), ((), ())),
                         precision=hi)
    dt = jnp.sqrt(b2_col + 2.0 * gt + a2_row)
    dti = lax.bitcast_convert_type(dt, jnp.int32)

    def body(_, c):
        lo2, hi2, lot, hit = c
        mid2 = lo2 + lax.shift_right_logical(hi2 - lo2, 1)
        midt = lot + lax.shift_right_logical(hit - lot, 1)
        cnt2 = jnp.sum((dti <= mid2).astype(jnp.int32), axis=0, keepdims=True)
        cntt = jnp.sum((dti <= midt).astype(jnp.int32), axis=0, keepdims=True)
        ok2 = cnt2 >= 2
        okt = cntt >= t
        return (jnp.where(ok2, lo2, mid2 + 1), jnp.where(ok2, mid2, hi2),
                jnp.where(okt, lot, midt + 1), jnp.where(okt, midt, hit))

    lo0 = jnp.zeros((1, _CH), jnp.int32)
    hi0 = jnp.full((1, _CH), _MAXF_BITS, jnp.int32)
    _, s2, _, st = lax.fori_loop(0, 31, body, (lo0, hi0, lo0, hi0))
    sigma = lax.bitcast_convert_type(s2, jnp.float32)
    vt = lax.bitcast_convert_type(st, jnp.float32)

    rr = dt / sigma
    wt = jnp.where(dt < vt, jnp.exp(-(rr * rr)), 0.0)
    wt2 = wt * wt
    rowsq = jnp.sum(wt2, axis=0, keepdims=True)
    colsq = lax.dot_general(ones_row, wt2, (((1,), (1,)), ((), ())),
                            precision=hi)
    ri = lax.broadcasted_iota(jnp.int32, (_CH, _CH), 0)
    ci = lax.broadcasted_iota(jnp.int32, (_CH, _CH), 1)
    wdiag = jnp.sum(jnp.where(ri == ci, wt, 0.0), axis=0, keepdims=True)
    f2 = colsq + rowsq
    eps = jnp.where(f2 > 0.0, wdiag / jnp.sqrt(jnp.maximum(f2, 1e-38)), 0.0)

    s_ref[...] = alpha * (1.0 + eps)


def _scale_kernel(s_ref, x_ref, o_ref):
    o_ref[...] = x_ref[...] * s_ref[...][0]


def kernel(x):
    b, c, h, w = x.shape
    hw = h * w
    xt = jnp.transpose(x, (0, 2, 3, 1)).reshape(b * hw, c)

    m = pl.pallas_call(
        _max_kernel,
        grid=(b, 2),
        in_specs=[pl.BlockSpec((hw // 2, c), lambda i, j: (i * 2 + j, 0))],
        out_specs=pl.BlockSpec((1, 1, c), lambda i, j: (i, 0, 0)),
        out_shape=jax.ShapeDtypeStruct((b, 1, c), jnp.float32),
    )(xt)

    scale = pl.pallas_call(
        _mid_kernel,
        out_shape=jax.ShapeDtypeStruct((b, c), jnp.float32),
    )(m.reshape(b, c))

    out = pl.pallas_call(
        _scale_kernel,
        grid=(b, 2),
        in_specs=[
            pl.BlockSpec((1, 1, c), lambda i, j: (i, 0, 0)),
            pl.BlockSpec((hw // 2, c), lambda i, j: (i * 2 + j, 0)),
        ],
        out_specs=pl.BlockSpec((hw // 2, c), lambda i, j: (i * 2 + j, 0)),
        out_shape=jax.ShapeDtypeStruct((b * hw, c), jnp.float32),
    )(scale.reshape(b, 1, c), xt)

    return jnp.transpose(out.reshape(b, h, w, c), (0, 3, 1, 2))


# speedup vs baseline: 1.1309x; 1.1016x over previous
import jax
import jax.numpy as jnp
from jax import lax
from jax.experimental import pallas as pl

_CH = 384
_B = 32
_MAXF_BITS = 0x7F7FFFFF


def _max_kernel(x_ref, m_ref):
    m_ref[0] = jnp.max(x_ref[...], axis=0, keepdims=True)


def _mid_kernel(m_ref, s_ref):
    m = m_ref[...]
    alpha = jax.nn.sigmoid(m)
    beta = 1.0 - alpha

    t = jnp.floor(jnp.sum(jnp.exp(beta - alpha)) / _B).astype(jnp.int32)
    t = jnp.where(t <= 2, 3, t)
    t = jnp.minimum(t, _CH)

    hi = lax.Precision.HIGHEST
    ones_col = jnp.ones((_B, 1), jnp.float32)
    ones_row = jnp.ones((1, _CH), jnp.float32)
    a2_row = jnp.sum(alpha * alpha, axis=0, keepdims=True)
    b2_col = lax.dot_general(beta * beta, ones_col, (((0,), (0,)), ((), ())),
                             precision=hi)
    gt = lax.dot_general(beta, alpha, (((0,), (0,)), ((), ())),
                         precision=hi)
    dt = jnp.sqrt(b2_col + 2.0 * gt + a2_row)
    dti = lax.bitcast_convert_type(dt, jnp.int32)

    def body(_, c):
        lo2, hi2, lot, hit = c
        mid2 = lo2 + lax.shift_right_logical(hi2 - lo2, 1)
        midt = lot + lax.shift_right_logical(hit - lot, 1)
        cnt2 = jnp.sum((dti <= mid2).astype(jnp.int32), axis=0, keepdims=True)
        cntt = jnp.sum((dti <= midt).astype(jnp.int32), axis=0, keepdims=True)
        ok2 = cnt2 >= 2
        okt = cntt >= t
        return (jnp.where(ok2, lo2, mid2 + 1), jnp.where(ok2, mid2, hi2),
                jnp.where(okt, lot, midt + 1), jnp.where(okt, midt, hit))

    lo0 = jnp.zeros((1, _CH), jnp.int32)
    hi0 = jnp.full((1, _CH), _MAXF_BITS, jnp.int32)
    _, s2, _, st = lax.fori_loop(0, 31, body, (lo0, hi0, lo0, hi0))
    sigma = lax.bitcast_convert_type(s2, jnp.float32)
    vt = lax.bitcast_convert_type(st, jnp.float32)

    rr = dt / sigma
    wt = jnp.where(dt < vt, jnp.exp(-(rr * rr)), 0.0)
    wt2 = wt * wt
    rowsq = jnp.sum(wt2, axis=0, keepdims=True)
    colsq = lax.dot_general(ones_row, wt2, (((1,), (1,)), ((), ())),
                            precision=hi)
    ri = lax.broadcasted_iota(jnp.int32, (_CH, _CH), 0)
    ci = lax.broadcasted_iota(jnp.int32, (_CH, _CH), 1)
    wdiag = jnp.sum(jnp.where(ri == ci, wt, 0.0), axis=0, keepdims=True)
    f2 = colsq + rowsq
    eps = jnp.where(f2 > 0.0, wdiag / jnp.sqrt(jnp.maximum(f2, 1e-38)), 0.0)

    s_ref[...] = alpha * (1.0 + eps)


def _scale_kernel(s_ref, x_ref, o_ref):
    o_ref[...] = x_ref[...] * s_ref[...][0]


def kernel(x):
    b, c, h, w = x.shape
    hw = h * w
    xt = jnp.transpose(x, (0, 2, 3, 1)).reshape(b * hw, c)

    m = pl.pallas_call(
        _max_kernel,
        grid=(b,),
        in_specs=[pl.BlockSpec((hw, c), lambda i: (i, 0))],
        out_specs=pl.BlockSpec((1, 1, c), lambda i: (i, 0, 0)),
        out_shape=jax.ShapeDtypeStruct((b, 1, c), jnp.float32),
    )(xt)

    scale = pl.pallas_call(
        _mid_kernel,
        out_shape=jax.ShapeDtypeStruct((b, c), jnp.float32),
    )(m.reshape(b, c))

    out = pl.pallas_call(
        _scale_kernel,
        grid=(b, 2),
        in_specs=[
            pl.BlockSpec((1, 1, c), lambda i, j: (i, 0, 0)),
            pl.BlockSpec((hw // 2, c), lambda i, j: (i * 2 + j, 0)),
        ],
        out_specs=pl.BlockSpec((hw // 2, c), lambda i, j: (i * 2 + j, 0)),
        out_shape=jax.ShapeDtypeStruct((b * hw, c), jnp.float32),
    )(scale.reshape(b, 1, c), xt)

    return jnp.transpose(out.reshape(b, h, w, c), (0, 3, 1, 2))


# speedup vs baseline: 1.1875x; 1.0500x over previous
import jax
import jax.numpy as jnp
from jax import lax
from jax.experimental import pallas as pl

_CH = 384
_B = 32
_MAXF_BITS = 0x7F7FFFFF


def _max_kernel(x_ref, m_ref):
    m_ref[0] = jnp.max(x_ref[...], axis=0, keepdims=True)


def _mid_kernel(m_ref, s_ref):
    m = m_ref[...]
    alpha = jax.nn.sigmoid(m)
    beta = 1.0 - alpha

    t = jnp.floor(jnp.sum(jnp.exp(beta - alpha)) / _B).astype(jnp.int32)
    t = jnp.where(t <= 2, 3, t)
    t = jnp.minimum(t, _CH)

    hi = lax.Precision.HIGHEST
    ones_col = jnp.ones((_B, 1), jnp.float32)
    ones_row = jnp.ones((1, _CH), jnp.float32)
    a2_row = jnp.sum(alpha * alpha, axis=0, keepdims=True)
    b2_col = lax.dot_general(beta * beta, ones_col, (((0,), (0,)), ((), ())),
                             precision=hi)
    gt = lax.dot_general(beta, alpha, (((0,), (0,)), ((), ())),
                         precision=hi)
    dt = jnp.sqrt(b2_col + 2.0 * gt + a2_row)
    dti = lax.bitcast_convert_type(dt, jnp.int32)

    def body(_, c):
        lo2, hi2, lot, hit = c
        mid2 = lo2 + lax.shift_right_logical(hi2 - lo2, 1)
        midt = lot + lax.shift_right_logical(hit - lot, 1)
        cnt2 = jnp.sum((dti <= mid2).astype(jnp.int32), axis=0, keepdims=True)
        cntt = jnp.sum((dti <= midt).astype(jnp.int32), axis=0, keepdims=True)
        ok2 = cnt2 >= 2
        okt = cntt >= t
        return (jnp.where(ok2, lo2, mid2 + 1), jnp.where(ok2, mid2, hi2),
                jnp.where(okt, lot, midt + 1), jnp.where(okt, midt, hit))

    lo0 = jnp.zeros((1, _CH), jnp.int32)
    hi0 = jnp.full((1, _CH), _MAXF_BITS, jnp.int32)
    _, s2, _, st = lax.fori_loop(0, 31, body, (lo0, hi0, lo0, hi0))
    sigma = lax.bitcast_convert_type(s2, jnp.float32)
    vt = lax.bitcast_convert_type(st, jnp.float32)

    rr = dt / sigma
    wt = jnp.where(dt < vt, jnp.exp(-(rr * rr)), 0.0)
    wt2 = wt * wt
    rowsq = jnp.sum(wt2, axis=0, keepdims=True)
    colsq = lax.dot_general(ones_row, wt2, (((1,), (1,)), ((), ())),
                            precision=hi)
    ri = lax.broadcasted_iota(jnp.int32, (_CH, _CH), 0)
    ci = lax.broadcasted_iota(jnp.int32, (_CH, _CH), 1)
    wdiag = jnp.sum(jnp.where(ri == ci, wt, 0.0), axis=0, keepdims=True)
    f2 = colsq + rowsq
    eps = jnp.where(f2 > 0.0, wdiag / jnp.sqrt(jnp.maximum(f2, 1e-38)), 0.0)

    s_ref[...] = alpha * (1.0 + eps)


def _scale_kernel(s_ref, x_ref, o_ref):
    s2 = s_ref[...]
    o_ref[0:3136, :] = x_ref[0:3136, :] * s2[0]
    o_ref[3136:6272, :] = x_ref[3136:6272, :] * s2[1]


def kernel(x):
    b, c, h, w = x.shape
    hw = h * w
    xt = jnp.transpose(x, (0, 2, 3, 1)).reshape(b * hw, c)

    m = pl.pallas_call(
        _max_kernel,
        grid=(b,),
        in_specs=[pl.BlockSpec((hw, c), lambda i: (i, 0))],
        out_specs=pl.BlockSpec((1, 1, c), lambda i: (i, 0, 0)),
        out_shape=jax.ShapeDtypeStruct((b, 1, c), jnp.float32),
    )(xt)

    scale = pl.pallas_call(
        _mid_kernel,
        out_shape=jax.ShapeDtypeStruct((b, c), jnp.float32),
    )(m.reshape(b, c))

    out = pl.pallas_call(
        _scale_kernel,
        grid=(b // 2,),
        in_specs=[
            pl.BlockSpec((2, 1, c), lambda i: (i, 0, 0)),
            pl.BlockSpec((2 * hw, c), lambda i: (i, 0)),
        ],
        out_specs=pl.BlockSpec((2 * hw, c), lambda i: (i, 0)),
        out_shape=jax.ShapeDtypeStruct((b * hw, c), jnp.float32),
    )(scale.reshape(b, 1, c), xt)

    return jnp.transpose(out.reshape(b, h, w, c), (0, 3, 1, 2))


# speedup vs baseline: 1.2063x; 1.0159x over previous
import jax
import jax.numpy as jnp
from jax import lax
from jax.experimental import pallas as pl

_CH = 384
_B = 32
_MAXF_BITS = 0x7F7FFFFF


def _max_kernel(x_ref, m_ref):
    x2 = x_ref[...]
    m_ref[0, 0:1] = jnp.max(x2[0:3136, :], axis=0, keepdims=True)
    m_ref[0, 1:2] = jnp.max(x2[3136:6272, :], axis=0, keepdims=True)


def _mid_kernel(m_ref, s_ref):
    m = m_ref[...]
    alpha = jax.nn.sigmoid(m)
    beta = 1.0 - alpha

    t = jnp.floor(jnp.sum(jnp.exp(beta - alpha)) / _B).astype(jnp.int32)
    t = jnp.where(t <= 2, 3, t)
    t = jnp.minimum(t, _CH)

    hi = lax.Precision.HIGHEST
    ones_col = jnp.ones((_B, 1), jnp.float32)
    ones_row = jnp.ones((1, _CH), jnp.float32)
    a2_row = jnp.sum(alpha * alpha, axis=0, keepdims=True)
    b2_col = lax.dot_general(beta * beta, ones_col, (((0,), (0,)), ((), ())),
                             precision=hi)
    gt = lax.dot_general(beta, alpha, (((0,), (0,)), ((), ())),
                         precision=hi)
    dt = jnp.sqrt(b2_col + 2.0 * gt + a2_row)
    dti = lax.bitcast_convert_type(dt, jnp.int32)

    def body(_, c):
        lo2, hi2, lot, hit = c
        mid2 = lo2 + lax.shift_right_logical(hi2 - lo2, 1)
        midt = lot + lax.shift_right_logical(hit - lot, 1)
        cnt2 = jnp.sum((dti <= mid2).astype(jnp.int32), axis=0, keepdims=True)
        cntt = jnp.sum((dti <= midt).astype(jnp.int32), axis=0, keepdims=True)
        ok2 = cnt2 >= 2
        okt = cntt >= t
        return (jnp.where(ok2, lo2, mid2 + 1), jnp.where(ok2, mid2, hi2),
                jnp.where(okt, lot, midt + 1), jnp.where(okt, midt, hit))

    lo0 = jnp.zeros((1, _CH), jnp.int32)
    hi0 = jnp.full((1, _CH), _MAXF_BITS, jnp.int32)
    _, s2, _, st = lax.fori_loop(0, 31, body, (lo0, hi0, lo0, hi0))
    sigma = lax.bitcast_convert_type(s2, jnp.float32)
    vt = lax.bitcast_convert_type(st, jnp.float32)

    rr = dt / sigma
    wt = jnp.where(dt < vt, jnp.exp(-(rr * rr)), 0.0)
    wt2 = wt * wt
    rowsq = jnp.sum(wt2, axis=0, keepdims=True)
    colsq = lax.dot_general(ones_row, wt2, (((1,), (1,)), ((), ())),
                            precision=hi)
    ri = lax.broadcasted_iota(jnp.int32, (_CH, _CH), 0)
    ci = lax.broadcasted_iota(jnp.int32, (_CH, _CH), 1)
    wdiag = jnp.sum(jnp.where(ri == ci, wt, 0.0), axis=0, keepdims=True)
    f2 = colsq + rowsq
    eps = jnp.where(f2 > 0.0, wdiag / jnp.sqrt(jnp.maximum(f2, 1e-38)), 0.0)

    s_ref[...] = alpha * (1.0 + eps)


def _scale_kernel(s_ref, x_ref, o_ref):
    s2 = s_ref[...]
    o_ref[0:3136, :] = x_ref[0:3136, :] * s2[0]
    o_ref[3136:6272, :] = x_ref[3136:6272, :] * s2[1]


def kernel(x):
    b, c, h, w = x.shape
    hw = h * w
    xt = jnp.transpose(x, (0, 2, 3, 1)).reshape(b * hw, c)

    m = pl.pallas_call(
        _max_kernel,
        grid=(b // 2,),
        in_specs=[pl.BlockSpec((2 * hw, c), lambda i: (i, 0))],
        out_specs=pl.BlockSpec((1, 2, c), lambda i: (i, 0, 0)),
        out_shape=jax.ShapeDtypeStruct((b // 2, 2, c), jnp.float32),
    )(xt)

    scale = pl.pallas_call(
        _mid_kernel,
        out_shape=jax.ShapeDtypeStruct((b, c), jnp.float32),
    )(m.reshape(b, c))

    out = pl.pallas_call(
        _scale_kernel,
        grid=(b // 2,),
        in_specs=[
            pl.BlockSpec((2, 1, c), lambda i: (i, 0, 0)),
            pl.BlockSpec((2 * hw, c), lambda i: (i, 0)),
        ],
        out_specs=pl.BlockSpec((2 * hw, c), lambda i: (i, 0)),
        out_shape=jax.ShapeDtypeStruct((b * hw, c), jnp.float32),
    )(scale.reshape(b, 1, c), xt)

    return jnp.transpose(out.reshape(b, h, w, c), (0, 3, 1, 2))
